# baseline (speedup 1.0000x reference)
import functools

import jax
import jax.numpy as jnp
from jax import lax
from jax.experimental import pallas as pl
from jax.experimental.pallas import tpu as pltpu
from jax.experimental.pallas import tpu_sc as plsc

N = 10000
E = 320000
DIN = 128
DH = 256
DOUT = 128
DE = 4
L = 3
G = 16

NW = 32
RT = 320
OUTR = NW * RT
ACC_R = RT + 8
CHUNK = 48
NGRP = CHUNK // 16
E_PAD = 323200
NF = DH // 16

BLK = 400
GRID_N = N // BLK

_f32 = jnp.float32


def _ln_rows(z, g, b):
    m = jnp.mean(z, axis=-1, keepdims=True)
    zc = z - m
    v = jnp.mean(zc * zc, axis=-1, keepdims=True)
    return zc * lax.rsqrt(v + 1e-5) * g + b


def _emb_body(x_ref, W_ref, b_ref, g_ref, be_ref, o_ref):
    z = jnp.dot(x_ref[:], W_ref[:], preferred_element_type=_f32) + b_ref[:]
    o_ref[:] = jnp.maximum(_ln_rows(z, g_ref[:], be_ref[:]), 0.0)


_emb = pl.pallas_call(
    _emb_body,
    grid=(GRID_N,),
    in_specs=[
        pl.BlockSpec((BLK, DIN), lambda i: (i, 0)),
        pl.BlockSpec((DIN, DH), lambda i: (0, 0)),
        pl.BlockSpec((1, DH), lambda i: (0, 0)),
        pl.BlockSpec((1, DH), lambda i: (0, 0)),
        pl.BlockSpec((1, DH), lambda i: (0, 0)),
    ],
    out_specs=pl.BlockSpec((BLK, DH), lambda i: (i, 0)),
    out_shape=jax.ShapeDtypeStruct((N, DH), _f32),
)


def _node_body(h_ref, nW1, nb1, ng, nbe, nW2, nb2, W1a, W1b,
               hn_ref, A_ref, B_ref):
    h = h_ref[:]
    z = jnp.dot(h, nW1[:], preferred_element_type=_f32) + nb1[:]
    h1 = jnp.maximum(_ln_rows(z, ng[:], nbe[:]), 0.0)
    hn_ref[:] = jnp.dot(h1, nW2[:], preferred_element_type=_f32) + nb2[:]
    A_ref[:] = jnp.dot(h, W1a[:], preferred_element_type=_f32)
    B_ref[:] = jnp.dot(h, W1b[:], preferred_element_type=_f32)


_mat_spec = pl.BlockSpec((DH, DH), lambda i: (0, 0))
_vec_spec = pl.BlockSpec((1, DH), lambda i: (0, 0))
_row_spec = pl.BlockSpec((BLK, DH), lambda i: (i, 0))

_node_stage = pl.pallas_call(
    _node_body,
    grid=(GRID_N,),
    in_specs=[_row_spec, _mat_spec, _vec_spec, _vec_spec, _vec_spec,
              _mat_spec, _vec_spec, _mat_spec, _mat_spec],
    out_specs=[_row_spec, _row_spec, _row_spec],
    out_shape=[jax.ShapeDtypeStruct((N, DH), _f32)] * 3,
)

EBLK = 3200
GRID_E = E_PAD // EBLK


def _c_body(ea_ref, Wc_ref, b1_ref, C_ref):
    C_ref[:] = (jnp.dot(ea_ref[:], Wc_ref[:], preferred_element_type=_f32)
                + b1_ref[:])


_c_stage = pl.pallas_call(
    _c_body,
    grid=(GRID_E,),
    in_specs=[
        pl.BlockSpec((EBLK, 8), lambda i: (i, 0)),
        pl.BlockSpec((8, DH), lambda i: (0, 0)),
        pl.BlockSpec((1, DH), lambda i: (0, 0)),
    ],
    out_specs=pl.BlockSpec((EBLK, DH), lambda i: (i, 0)),
    out_shape=jax.ShapeDtypeStruct((E_PAD, DH), _f32),
)


def _upd_body(S_ref, dega_ref, hn_ref, hp_ref, eW2, U1, U2, ub, lg, lb, o_ref):
    h_agg = jnp.dot(S_ref[:], eW2[:], preferred_element_type=_f32) + dega_ref[:]
    u = (jnp.dot(hn_ref[:], U1[:], preferred_element_type=_f32)
         + jnp.dot(h_agg, U2[:], preferred_element_type=_f32)
         + ub[:] + hp_ref[:])
    o_ref[:] = _ln_rows(u, lg[:], lb[:])


_upd_stage = pl.pallas_call(
    _upd_body,
    grid=(GRID_N,),
    in_specs=[_row_spec, _row_spec, _row_spec, _row_spec,
              _mat_spec, _mat_spec, _mat_spec, _vec_spec, _vec_spec, _vec_spec],
    out_specs=_row_spec,
    out_shape=jax.ShapeDtypeStruct((N, DH), _f32),
)


def _out_body(h_ref, W, b, g, be, o_ref):
    z = jnp.dot(h_ref[:], W[:], preferred_element_type=_f32) + b[:]
    o_ref[:] = _ln_rows(z, g[:], be[:])


_out_stage = pl.pallas_call(
    _out_body,
    grid=(GRID_N,),
    in_specs=[
        _row_spec,
        pl.BlockSpec((DH, DOUT), lambda i: (0, 0)),
        pl.BlockSpec((1, DOUT), lambda i: (0, 0)),
        pl.BlockSpec((1, DOUT), lambda i: (0, 0)),
        pl.BlockSpec((1, DOUT), lambda i: (0, 0)),
    ],
    out_specs=pl.BlockSpec((BLK, DOUT), lambda i: (i, 0)),
    out_shape=jax.ShapeDtypeStruct((N, DOUT), _f32),
)


def _pool_body(ne_ref, b_ref, Cw1, Cw2, cb, o_ref, acc, cacc):
    pid = pl.program_id(0)

    @pl.when(pid == 0)
    def _():
        acc[:] = jnp.zeros_like(acc)
        cacc[:] = jnp.zeros_like(cacc)

    bb = b_ref[0, 0, :]
    oh = (lax.broadcasted_iota(jnp.int32, (G, BLK), 0) == bb[None, :])
    oh = oh.astype(_f32)
    acc[:] += jnp.dot(oh, ne_ref[:], preferred_element_type=_f32)
    cacc[:] += jnp.broadcast_to(jnp.sum(oh, axis=1, keepdims=True), (G, DOUT))

    @pl.when(pid == GRID_N - 1)
    def _():
        a = acc[:]
        c = jnp.maximum(cacc[:], 1.0)
        o_ref[:] = (jnp.dot(a / c, Cw1[:], preferred_element_type=_f32)
                    + jnp.dot(a, Cw2[:], preferred_element_type=_f32) + cb[:])


_pool_stage = pl.pallas_call(
    _pool_body,
    grid=(GRID_N,),
    in_specs=[
        pl.BlockSpec((BLK, DOUT), lambda i: (i, 0)),
        pl.BlockSpec((1, 1, BLK), lambda i: (i, 0, 0)),
        pl.BlockSpec((DOUT, DOUT), lambda i: (0, 0)),
        pl.BlockSpec((DOUT, DOUT), lambda i: (0, 0)),
        pl.BlockSpec((1, DOUT), lambda i: (0, 0)),
    ],
    out_specs=pl.BlockSpec((G, DOUT), lambda i: (0, 0)),
    out_shape=jax.ShapeDtypeStruct((G, DOUT), _f32),
    scratch_shapes=[pltpu.VMEM((G, DOUT), _f32), pltpu.VMEM((G, DOUT), _f32)],
)


_sc_mesh = plsc.VectorSubcoreMesh(core_axis_name="c", subcore_axis_name="s")


_GDN = lax.GatherDimensionNumbers(
    offset_dims=(), collapsed_slice_dims=(0,), start_index_map=(0,))


def _lanesum(v):
    lanes = lax.iota(jnp.int32, 16)
    for d in (1, 2, 4, 8):
        idx = lax.bitwise_xor(lanes, jnp.full((16,), d, jnp.int32))
        v = v + lax.gather(v, idx[:, None], _GDN, (1,),
                           mode=lax.GatherScatterMode.PROMISE_IN_BOUNDS)
    return v


def _rsqrt16(x):
    s = 0.5 * (x + 1.0)
    for _ in range(14):
        s = 0.5 * (s + x / s)
    return 1.0 / s


@functools.partial(
    pl.kernel,
    mesh=_sc_mesh,
    out_type=jax.ShapeDtypeStruct((OUTR, DH), _f32),
    scratch_types=[
        pltpu.VMEM((16,), jnp.int32),
        pltpu.VMEM((CHUNK,), jnp.int32),
        pltpu.VMEM((CHUNK,), jnp.int32),
        pltpu.VMEM((CHUNK, DH), _f32),
        pltpu.VMEM((CHUNK, DH), _f32),
        pltpu.VMEM((CHUNK, DH), _f32),
        pltpu.VMEM((8, DH), _f32),
        pltpu.VMEM((2, DH), _f32),
        pltpu.VMEM((ACC_R, DH), _f32),
        pltpu.SemaphoreType.DMA,
        pltpu.SemaphoreType.DMA,
    ],
)
def _edge_sc(A_hbm, B_hbm, C_hbm, src_hbm, dst_hbm, bnds_hbm, gbe_hbm,
             out_hbm, bv, src_v, dst_v, Av, Bv, Cv, tbuf, gbev, acc,
             semA, semB):
    cid = lax.axis_index("c")
    sid = lax.axis_index("s")
    w = cid * 16 + sid

    pltpu.sync_copy(gbe_hbm, gbev)
    pltpu.sync_copy(bnds_hbm.at[pl.ds(w * 16, 16)], bv)
    bvec = bv[pl.ds(0, 16)]
    lo = bvec[0]
    hi = bvec[1]
    loa = (lo // 8) * 8
    niter = (hi - loa + (CHUNK - 1)) // CHUNK

    zero16 = jnp.zeros((16,), _f32)

    def _zrow(r, _):
        for k in range(NF):
            acc[r, pl.ds(k * 16, 16)] = zero16
        return 0

    lax.fori_loop(0, ACC_R, _zrow, 0)

    wbase = lax.broadcast_in_dim(w * RT, (16,), ())

    def _group(g, _):
        dvec = dst_v[pl.ds(g * 16, 16)] - wbase
        for j in range(16):
            e = g * 16 + j
            rl = dvec[j]
            ok = (rl >= 0) & (rl < RT)
            rl = jnp.where(ok, rl, RT)
            acc_s = jnp.zeros((16,), _f32)
            acc_q = jnp.zeros((16,), _f32)
            for k in range(NF):
                sl = pl.ds(k * 16, 16)
                t = Av[e, sl] + Bv[e, sl] + Cv[e, sl]
                tbuf[0, sl] = t
                acc_s = acc_s + t
                acc_q = acc_q + t * t
            mv = _lanesum(acc_s) * (1.0 / DH)
            vv = _lanesum(acc_q) * (1.0 / DH) - mv * mv
            vv = jnp.maximum(vv, 0.0) + 1e-5
            inv = _rsqrt16(vv)
            for k in range(NF):
                sl = pl.ds(k * 16, 16)
                y = (tbuf[0, sl] - mv) * inv
                y = jnp.maximum(y * gbev[0, sl] + gbev[1, sl], 0.0)
                acc[rl, sl] = acc[rl, sl] + y
        return 0

    def _step(jc, _):
        off = loa + jc * CHUNK
        pltpu.sync_copy(src_hbm.at[pl.ds(off, CHUNK)], src_v)
        pltpu.sync_copy(dst_hbm.at[pl.ds(off, CHUNK)], dst_v)
        cpa = pltpu.async_copy(A_hbm.at[src_v], Av, semA)
        cpb = pltpu.async_copy(B_hbm.at[dst_v], Bv, semB)
        pltpu.sync_copy(C_hbm.at[pl.ds(off, CHUNK)], Cv)
        cpa.wait()
        cpb.wait()
        lax.fori_loop(0, NGRP, _group, 0)
        return 0

    lax.fori_loop(0, niter, _step, 0)
    pltpu.sync_copy(acc.at[pl.ds(0, RT)], out_hbm.at[pl.ds(w * RT, RT)])


def kernel(x, edge_index, edge_attr, pos, batch, emb_W, emb_b, emb_g, emb_be,
           node_W1, node_b1, node_g, node_be, node_W2, node_b2,
           edge_W1, edge_b1, edge_g, edge_be, edge_W2, edge_b2,
           upd_W, upd_b, ln_g, ln_be,
           out_W, out_b, out_g, out_be, comb_W, comb_b):
    src = edge_index[0]
    dst = edge_index[1]
    deg = jnp.zeros((N,), _f32).at[dst].add(1.0)

    order = jnp.argsort(dst)
    dst_s = jnp.pad(dst[order], (0, E_PAD - E), constant_values=N + 512)
    src_s = jnp.pad(src[order], (0, E_PAD - E))
    ea_s = jnp.pad(edge_attr[order], ((0, E_PAD - E), (0, 8 - DE)))
    bounds = jnp.searchsorted(dst_s[:E], jnp.arange(NW + 1, dtype=jnp.int32)
                              * RT).astype(jnp.int32)
    bnds = jnp.zeros((NW, 16), jnp.int32)
    bnds = bnds.at[:, 0].set(bounds[:NW]).at[:, 1].set(bounds[1:])
    bnds = bnds.reshape(-1)
    r1 = lambda v: v.reshape(1, -1)

    h = _emb(x, emb_W, r1(emb_b), r1(emb_g), r1(emb_be))
    for i in range(L):
        W1 = edge_W1[i]
        hn, A, B = _node_stage(h, node_W1[i], r1(node_b1[i]), r1(node_g[i]),
                               r1(node_be[i]), node_W2[i], r1(node_b2[i]),
                               W1[:DH], W1[DH:2 * DH])
        Wc = jnp.pad(W1[2 * DH:], ((0, 8 - DE), (0, 0)))
        C = _c_stage(ea_s, Wc, r1(edge_b1[i]))
        gbe = jnp.stack([edge_g[i], edge_be[i]])
        Sp = _edge_sc(A, B, C, src_s, dst_s, bnds, gbe)
        S = Sp[:N]
        dega = deg[:, None] * edge_b2[i][None, :]
        h = _upd_stage(S, dega, hn, h, edge_W2[i], upd_W[i][:DH],
                       upd_W[i][DH:], r1(upd_b[i]), r1(ln_g[i]), r1(ln_be[i]))

    node_emb = _out_stage(h, out_W, r1(out_b), r1(out_g), r1(out_be))
    graph_emb = _pool_stage(node_emb, batch.reshape(GRID_N, 1, BLK),
                            comb_W[:DOUT], comb_W[DOUT:], r1(comb_b))
    return node_emb, graph_emb


# speedup vs baseline: 1.2450x; 1.2450x over previous
import functools

import jax
import jax.numpy as jnp
from jax import lax
from jax.experimental import pallas as pl
from jax.experimental.pallas import tpu as pltpu
from jax.experimental.pallas import tpu_sc as plsc

N = 10000
E = 320000
DIN = 128
DH = 256
DOUT = 128
DE = 4
L = 3
G = 16

NW = 32
RT = 320
OUTR = NW * RT
ACC_R = RT + 8
CHUNK = 48
NGRP = CHUNK // 16
E_PAD = 323200
NF = DH // 16

BLK = 400
GRID_N = N // BLK

_f32 = jnp.float32


def _ln_rows(z, g, b):
    m = jnp.mean(z, axis=-1, keepdims=True)
    zc = z - m
    v = jnp.mean(zc * zc, axis=-1, keepdims=True)
    return zc * lax.rsqrt(v + 1e-5) * g + b


def _emb_body(x_ref, W_ref, b_ref, g_ref, be_ref, o_ref):
    z = jnp.dot(x_ref[:], W_ref[:], preferred_element_type=_f32) + b_ref[:]
    o_ref[:] = jnp.maximum(_ln_rows(z, g_ref[:], be_ref[:]), 0.0)


_emb = pl.pallas_call(
    _emb_body,
    grid=(GRID_N,),
    in_specs=[
        pl.BlockSpec((BLK, DIN), lambda i: (i, 0)),
        pl.BlockSpec((DIN, DH), lambda i: (0, 0)),
        pl.BlockSpec((1, DH), lambda i: (0, 0)),
        pl.BlockSpec((1, DH), lambda i: (0, 0)),
        pl.BlockSpec((1, DH), lambda i: (0, 0)),
    ],
    out_specs=pl.BlockSpec((BLK, DH), lambda i: (i, 0)),
    out_shape=jax.ShapeDtypeStruct((N, DH), _f32),
)


def _node_body(h_ref, nW1, nb1, ng, nbe, nW2, nb2, W1a, W1b,
               hn_ref, A_ref, B_ref):
    h = h_ref[:]
    z = jnp.dot(h, nW1[:], preferred_element_type=_f32) + nb1[:]
    h1 = jnp.maximum(_ln_rows(z, ng[:], nbe[:]), 0.0)
    hn_ref[:] = jnp.dot(h1, nW2[:], preferred_element_type=_f32) + nb2[:]
    A_ref[:] = jnp.dot(h, W1a[:], preferred_element_type=_f32)
    B_ref[:] = jnp.dot(h, W1b[:], preferred_element_type=_f32)


_mat_spec = pl.BlockSpec((DH, DH), lambda i: (0, 0))
_vec_spec = pl.BlockSpec((1, DH), lambda i: (0, 0))
_row_spec = pl.BlockSpec((BLK, DH), lambda i: (i, 0))

_node_stage = pl.pallas_call(
    _node_body,
    grid=(GRID_N,),
    in_specs=[_row_spec, _mat_spec, _vec_spec, _vec_spec, _vec_spec,
              _mat_spec, _vec_spec, _mat_spec, _mat_spec],
    out_specs=[_row_spec, _row_spec, _row_spec],
    out_shape=[jax.ShapeDtypeStruct((N, DH), _f32)] * 3,
)

EBLK = 3200
GRID_E = E_PAD // EBLK


def _c_body(ea_ref, Wc_ref, b1_ref, C_ref):
    C_ref[:] = (jnp.dot(ea_ref[:], Wc_ref[:], preferred_element_type=_f32)
                + b1_ref[:])


_c_stage = pl.pallas_call(
    _c_body,
    grid=(GRID_E,),
    in_specs=[
        pl.BlockSpec((EBLK, 8), lambda i: (i, 0)),
        pl.BlockSpec((8, DH), lambda i: (0, 0)),
        pl.BlockSpec((1, DH), lambda i: (0, 0)),
    ],
    out_specs=pl.BlockSpec((EBLK, DH), lambda i: (i, 0)),
    out_shape=jax.ShapeDtypeStruct((E_PAD, DH), _f32),
)


def _upd_body(S_ref, dega_ref, hn_ref, hp_ref, eW2, U1, U2, ub, lg, lb, o_ref):
    h_agg = jnp.dot(S_ref[:], eW2[:], preferred_element_type=_f32) + dega_ref[:]
    u = (jnp.dot(hn_ref[:], U1[:], preferred_element_type=_f32)
         + jnp.dot(h_agg, U2[:], preferred_element_type=_f32)
         + ub[:] + hp_ref[:])
    o_ref[:] = _ln_rows(u, lg[:], lb[:])


_upd_stage = pl.pallas_call(
    _upd_body,
    grid=(GRID_N,),
    in_specs=[_row_spec, _row_spec, _row_spec, _row_spec,
              _mat_spec, _mat_spec, _mat_spec, _vec_spec, _vec_spec, _vec_spec],
    out_specs=_row_spec,
    out_shape=jax.ShapeDtypeStruct((N, DH), _f32),
)


def _out_body(h_ref, W, b, g, be, o_ref):
    z = jnp.dot(h_ref[:], W[:], preferred_element_type=_f32) + b[:]
    o_ref[:] = _ln_rows(z, g[:], be[:])


_out_stage = pl.pallas_call(
    _out_body,
    grid=(GRID_N,),
    in_specs=[
        _row_spec,
        pl.BlockSpec((DH, DOUT), lambda i: (0, 0)),
        pl.BlockSpec((1, DOUT), lambda i: (0, 0)),
        pl.BlockSpec((1, DOUT), lambda i: (0, 0)),
        pl.BlockSpec((1, DOUT), lambda i: (0, 0)),
    ],
    out_specs=pl.BlockSpec((BLK, DOUT), lambda i: (i, 0)),
    out_shape=jax.ShapeDtypeStruct((N, DOUT), _f32),
)


def _pool_body(ne_ref, b_ref, Cw1, Cw2, cb, o_ref, acc, cacc):
    pid = pl.program_id(0)

    @pl.when(pid == 0)
    def _():
        acc[:] = jnp.zeros_like(acc)
        cacc[:] = jnp.zeros_like(cacc)

    bb = b_ref[0, 0, :]
    oh = (lax.broadcasted_iota(jnp.int32, (G, BLK), 0) == bb[None, :])
    oh = oh.astype(_f32)
    acc[:] += jnp.dot(oh, ne_ref[:], preferred_element_type=_f32)
    cacc[:] += jnp.broadcast_to(jnp.sum(oh, axis=1, keepdims=True), (G, DOUT))

    @pl.when(pid == GRID_N - 1)
    def _():
        a = acc[:]
        c = jnp.maximum(cacc[:], 1.0)
        o_ref[:] = (jnp.dot(a / c, Cw1[:], preferred_element_type=_f32)
                    + jnp.dot(a, Cw2[:], preferred_element_type=_f32) + cb[:])


_pool_stage = pl.pallas_call(
    _pool_body,
    grid=(GRID_N,),
    in_specs=[
        pl.BlockSpec((BLK, DOUT), lambda i: (i, 0)),
        pl.BlockSpec((1, 1, BLK), lambda i: (i, 0, 0)),
        pl.BlockSpec((DOUT, DOUT), lambda i: (0, 0)),
        pl.BlockSpec((DOUT, DOUT), lambda i: (0, 0)),
        pl.BlockSpec((1, DOUT), lambda i: (0, 0)),
    ],
    out_specs=pl.BlockSpec((G, DOUT), lambda i: (0, 0)),
    out_shape=jax.ShapeDtypeStruct((G, DOUT), _f32),
    scratch_shapes=[pltpu.VMEM((G, DOUT), _f32), pltpu.VMEM((G, DOUT), _f32)],
)


_sc_mesh = plsc.VectorSubcoreMesh(core_axis_name="c", subcore_axis_name="s")


_GDN = lax.GatherDimensionNumbers(
    offset_dims=(), collapsed_slice_dims=(0,), start_index_map=(0,))


def _lanesum(v):
    lanes = lax.iota(jnp.int32, 16)
    for d in (1, 2, 4, 8):
        idx = lax.bitwise_xor(lanes, jnp.full((16,), d, jnp.int32))
        v = v + lax.gather(v, idx[:, None], _GDN, (1,),
                           mode=lax.GatherScatterMode.PROMISE_IN_BOUNDS)
    return v


def _rsqrt16(x):
    s = 0.5 * (x + 1.0)
    for _ in range(14):
        s = 0.5 * (s + x / s)
    return 1.0 / s


@functools.partial(
    pl.kernel,
    mesh=_sc_mesh,
    out_type=jax.ShapeDtypeStruct((OUTR, DH), _f32),
    scratch_types=[
        pltpu.VMEM((16,), jnp.int32),
        pltpu.VMEM((CHUNK,), jnp.int32),
        pltpu.VMEM((CHUNK,), jnp.int32),
        pltpu.VMEM((CHUNK, DH), _f32),
        pltpu.VMEM((CHUNK, DH), _f32),
        pltpu.VMEM((CHUNK, DH), _f32),
        pltpu.VMEM((16, DH), _f32),
        pltpu.VMEM((2, DH), _f32),
        pltpu.VMEM((ACC_R, DH), _f32),
        pltpu.SemaphoreType.DMA,
        pltpu.SemaphoreType.DMA,
    ],
)
def _edge_sc(A_hbm, B_hbm, C_hbm, src_hbm, dst_hbm, bnds_hbm, gbe_hbm,
             out_hbm, bv, src_v, dst_v, Av, Bv, Cv, tbuf, gbev, acc,
             semA, semB):
    cid = lax.axis_index("c")
    sid = lax.axis_index("s")
    w = cid * 16 + sid

    pltpu.sync_copy(gbe_hbm, gbev)
    pltpu.sync_copy(bnds_hbm.at[pl.ds(w * 16, 16)], bv)
    bvec = bv[pl.ds(0, 16)]
    lo = bvec[0]
    hi = bvec[1]
    loa = (lo // 8) * 8
    niter = (hi - loa + (CHUNK - 1)) // CHUNK

    zero16 = jnp.zeros((16,), _f32)

    def _zrow(r, _):
        for k in range(NF):
            acc[r, pl.ds(k * 16, 16)] = zero16
        return 0

    lax.fori_loop(0, ACC_R, _zrow, 0)

    wbase = lax.broadcast_in_dim(w * RT, (16,), ())

    lane = lax.iota(jnp.int32, 16)

    def _group(g, _):
        dvec = dst_v[pl.ds(g * 16, 16)] - wbase
        svec = jnp.zeros((16,), _f32)
        qvec = jnp.zeros((16,), _f32)
        for j in range(16):
            e = g * 16 + j
            acc_s = jnp.zeros((16,), _f32)
            acc_q = jnp.zeros((16,), _f32)
            for k in range(NF):
                sl = pl.ds(k * 16, 16)
                t = Av[e, sl] + Bv[e, sl] + Cv[e, sl]
                tbuf[j, sl] = t
                acc_s = acc_s + t
                acc_q = acc_q + t * t
            jm = lane == j
            svec = jnp.where(jm, _lanesum(acc_s), svec)
            qvec = jnp.where(jm, _lanesum(acc_q), qvec)
        mvec = svec * (1.0 / DH)
        vvec = jnp.maximum(qvec * (1.0 / DH) - mvec * mvec, 0.0) + 1e-5
        ivec = _rsqrt16(vvec)
        for j in range(16):
            rl = dvec[j]
            ok = (rl >= 0) & (rl < RT)
            rl = jnp.where(ok, rl, RT)
            jidx = jnp.full((16,), j, jnp.int32)
            mj = lax.gather(mvec, jidx[:, None], _GDN, (1,),
                            mode=lax.GatherScatterMode.PROMISE_IN_BOUNDS)
            ij = lax.gather(ivec, jidx[:, None], _GDN, (1,),
                            mode=lax.GatherScatterMode.PROMISE_IN_BOUNDS)
            for k in range(NF):
                sl = pl.ds(k * 16, 16)
                y = (tbuf[j, sl] - mj) * ij
                y = jnp.maximum(y * gbev[0, sl] + gbev[1, sl], 0.0)
                acc[rl, sl] = acc[rl, sl] + y
        return 0

    def _step(jc, _):
        off = loa + jc * CHUNK
        pltpu.sync_copy(src_hbm.at[pl.ds(off, CHUNK)], src_v)
        pltpu.sync_copy(dst_hbm.at[pl.ds(off, CHUNK)], dst_v)
        cpa = pltpu.async_copy(A_hbm.at[src_v], Av, semA)
        cpb = pltpu.async_copy(B_hbm.at[dst_v], Bv, semB)
        pltpu.sync_copy(C_hbm.at[pl.ds(off, CHUNK)], Cv)
        cpa.wait()
        cpb.wait()
        lax.fori_loop(0, NGRP, _group, 0)
        return 0

    lax.fori_loop(0, niter, _step, 0)
    pltpu.sync_copy(acc.at[pl.ds(0, RT)], out_hbm.at[pl.ds(w * RT, RT)])


def kernel(x, edge_index, edge_attr, pos, batch, emb_W, emb_b, emb_g, emb_be,
           node_W1, node_b1, node_g, node_be, node_W2, node_b2,
           edge_W1, edge_b1, edge_g, edge_be, edge_W2, edge_b2,
           upd_W, upd_b, ln_g, ln_be,
           out_W, out_b, out_g, out_be, comb_W, comb_b):
    src = edge_index[0]
    dst = edge_index[1]
    deg = jnp.zeros((N,), _f32).at[dst].add(1.0)

    order = jnp.argsort(dst)
    dst_s = jnp.pad(dst[order], (0, E_PAD - E), constant_values=N + 512)
    src_s = jnp.pad(src[order], (0, E_PAD - E))
    ea_s = jnp.pad(edge_attr[order], ((0, E_PAD - E), (0, 8 - DE)))
    bounds = jnp.searchsorted(dst_s[:E], jnp.arange(NW + 1, dtype=jnp.int32)
                              * RT).astype(jnp.int32)
    bnds = jnp.zeros((NW, 16), jnp.int32)
    bnds = bnds.at[:, 0].set(bounds[:NW]).at[:, 1].set(bounds[1:])
    bnds = bnds.reshape(-1)
    r1 = lambda v: v.reshape(1, -1)

    h = _emb(x, emb_W, r1(emb_b), r1(emb_g), r1(emb_be))
    for i in range(L):
        W1 = edge_W1[i]
        hn, A, B = _node_stage(h, node_W1[i], r1(node_b1[i]), r1(node_g[i]),
                               r1(node_be[i]), node_W2[i], r1(node_b2[i]),
                               W1[:DH], W1[DH:2 * DH])
        Wc = jnp.pad(W1[2 * DH:], ((0, 8 - DE), (0, 0)))
        C = _c_stage(ea_s, Wc, r1(edge_b1[i]))
        gbe = jnp.stack([edge_g[i], edge_be[i]])
        Sp = _edge_sc(A, B, C, src_s, dst_s, bnds, gbe)
        S = Sp[:N]
        dega = deg[:, None] * edge_b2[i][None, :]
        h = _upd_stage(S, dega, hn, h, edge_W2[i], upd_W[i][:DH],
                       upd_W[i][DH:], r1(upd_b[i]), r1(ln_g[i]), r1(ln_be[i]))

    node_emb = _out_stage(h, out_W, r1(out_b), r1(out_g), r1(out_be))
    graph_emb = _pool_stage(node_emb, batch.reshape(GRID_N, 1, BLK),
                            comb_W[:DOUT], comb_W[DOUT:], r1(comb_b))
    return node_emb, graph_emb


# speedup vs baseline: 1.2680x; 1.0185x over previous
import functools

import jax
import jax.numpy as jnp
from jax import lax
from jax.experimental import pallas as pl
from jax.experimental.pallas import tpu as pltpu
from jax.experimental.pallas import tpu_sc as plsc

N = 10000
E = 320000
DIN = 128
DH = 256
DOUT = 128
DE = 4
L = 3
G = 16

NW = 32
RT = 320
OUTR = NW * RT
ACC_R = RT + 8
CHUNK = 48
NGRP = CHUNK // 16
E_PAD = 323200
NF = DH // 16

BLK = 400
GRID_N = N // BLK

_f32 = jnp.float32


def _ln_rows(z, g, b):
    m = jnp.mean(z, axis=-1, keepdims=True)
    zc = z - m
    v = jnp.mean(zc * zc, axis=-1, keepdims=True)
    return zc * lax.rsqrt(v + 1e-5) * g + b


def _emb_body(x_ref, W_ref, b_ref, g_ref, be_ref, o_ref):
    z = jnp.dot(x_ref[:], W_ref[:], preferred_element_type=_f32) + b_ref[:]
    o_ref[:] = jnp.maximum(_ln_rows(z, g_ref[:], be_ref[:]), 0.0)


_emb = pl.pallas_call(
    _emb_body,
    grid=(GRID_N,),
    in_specs=[
        pl.BlockSpec((BLK, DIN), lambda i: (i, 0)),
        pl.BlockSpec((DIN, DH), lambda i: (0, 0)),
        pl.BlockSpec((1, DH), lambda i: (0, 0)),
        pl.BlockSpec((1, DH), lambda i: (0, 0)),
        pl.BlockSpec((1, DH), lambda i: (0, 0)),
    ],
    out_specs=pl.BlockSpec((BLK, DH), lambda i: (i, 0)),
    out_shape=jax.ShapeDtypeStruct((N, DH), _f32),
)


def _node_body(h_ref, nW1, nb1, ng, nbe, nW2, nb2, W1a, W1b,
               hn_ref, A_ref, B_ref):
    h = h_ref[:]
    z = jnp.dot(h, nW1[:], preferred_element_type=_f32) + nb1[:]
    h1 = jnp.maximum(_ln_rows(z, ng[:], nbe[:]), 0.0)
    hn_ref[:] = jnp.dot(h1, nW2[:], preferred_element_type=_f32) + nb2[:]
    A_ref[:] = jnp.dot(h, W1a[:], preferred_element_type=_f32)
    B_ref[:] = jnp.dot(h, W1b[:], preferred_element_type=_f32)


_mat_spec = pl.BlockSpec((DH, DH), lambda i: (0, 0))
_vec_spec = pl.BlockSpec((1, DH), lambda i: (0, 0))
_row_spec = pl.BlockSpec((BLK, DH), lambda i: (i, 0))

_node_stage = pl.pallas_call(
    _node_body,
    grid=(GRID_N,),
    in_specs=[_row_spec, _mat_spec, _vec_spec, _vec_spec, _vec_spec,
              _mat_spec, _vec_spec, _mat_spec, _mat_spec],
    out_specs=[_row_spec, _row_spec, _row_spec],
    out_shape=[jax.ShapeDtypeStruct((N, DH), _f32)] * 3,
)

EBLK = 3200
GRID_E = E_PAD // EBLK


def _c_body(ea_ref, Wc_ref, b1_ref, C_ref):
    C_ref[:] = (jnp.dot(ea_ref[:], Wc_ref[:], preferred_element_type=_f32)
                + b1_ref[:])


_c_stage = pl.pallas_call(
    _c_body,
    grid=(GRID_E,),
    in_specs=[
        pl.BlockSpec((EBLK, 8), lambda i: (i, 0)),
        pl.BlockSpec((8, DH), lambda i: (0, 0)),
        pl.BlockSpec((1, DH), lambda i: (0, 0)),
    ],
    out_specs=pl.BlockSpec((EBLK, DH), lambda i: (i, 0)),
    out_shape=jax.ShapeDtypeStruct((E_PAD, DH), _f32),
)


def _upd_body(S_ref, dega_ref, hn_ref, hp_ref, eW2, U1, U2, ub, lg, lb, o_ref):
    h_agg = jnp.dot(S_ref[:], eW2[:], preferred_element_type=_f32) + dega_ref[:]
    u = (jnp.dot(hn_ref[:], U1[:], preferred_element_type=_f32)
         + jnp.dot(h_agg, U2[:], preferred_element_type=_f32)
         + ub[:] + hp_ref[:])
    o_ref[:] = _ln_rows(u, lg[:], lb[:])


_upd_stage = pl.pallas_call(
    _upd_body,
    grid=(GRID_N,),
    in_specs=[_row_spec, _row_spec, _row_spec, _row_spec,
              _mat_spec, _mat_spec, _mat_spec, _vec_spec, _vec_spec, _vec_spec],
    out_specs=_row_spec,
    out_shape=jax.ShapeDtypeStruct((N, DH), _f32),
)


def _out_body(h_ref, W, b, g, be, o_ref):
    z = jnp.dot(h_ref[:], W[:], preferred_element_type=_f32) + b[:]
    o_ref[:] = _ln_rows(z, g[:], be[:])


_out_stage = pl.pallas_call(
    _out_body,
    grid=(GRID_N,),
    in_specs=[
        _row_spec,
        pl.BlockSpec((DH, DOUT), lambda i: (0, 0)),
        pl.BlockSpec((1, DOUT), lambda i: (0, 0)),
        pl.BlockSpec((1, DOUT), lambda i: (0, 0)),
        pl.BlockSpec((1, DOUT), lambda i: (0, 0)),
    ],
    out_specs=pl.BlockSpec((BLK, DOUT), lambda i: (i, 0)),
    out_shape=jax.ShapeDtypeStruct((N, DOUT), _f32),
)


def _pool_body(ne_ref, b_ref, Cw1, Cw2, cb, o_ref, acc, cacc):
    pid = pl.program_id(0)

    @pl.when(pid == 0)
    def _():
        acc[:] = jnp.zeros_like(acc)
        cacc[:] = jnp.zeros_like(cacc)

    bb = b_ref[0, 0, :]
    oh = (lax.broadcasted_iota(jnp.int32, (G, BLK), 0) == bb[None, :])
    oh = oh.astype(_f32)
    acc[:] += jnp.dot(oh, ne_ref[:], preferred_element_type=_f32)
    cacc[:] += jnp.broadcast_to(jnp.sum(oh, axis=1, keepdims=True), (G, DOUT))

    @pl.when(pid == GRID_N - 1)
    def _():
        a = acc[:]
        c = jnp.maximum(cacc[:], 1.0)
        o_ref[:] = (jnp.dot(a / c, Cw1[:], preferred_element_type=_f32)
                    + jnp.dot(a, Cw2[:], preferred_element_type=_f32) + cb[:])


_pool_stage = pl.pallas_call(
    _pool_body,
    grid=(GRID_N,),
    in_specs=[
        pl.BlockSpec((BLK, DOUT), lambda i: (i, 0)),
        pl.BlockSpec((1, 1, BLK), lambda i: (i, 0, 0)),
        pl.BlockSpec((DOUT, DOUT), lambda i: (0, 0)),
        pl.BlockSpec((DOUT, DOUT), lambda i: (0, 0)),
        pl.BlockSpec((1, DOUT), lambda i: (0, 0)),
    ],
    out_specs=pl.BlockSpec((G, DOUT), lambda i: (0, 0)),
    out_shape=jax.ShapeDtypeStruct((G, DOUT), _f32),
    scratch_shapes=[pltpu.VMEM((G, DOUT), _f32), pltpu.VMEM((G, DOUT), _f32)],
)


_sc_mesh = plsc.VectorSubcoreMesh(core_axis_name="c", subcore_axis_name="s")


_GDN = lax.GatherDimensionNumbers(
    offset_dims=(), collapsed_slice_dims=(0,), start_index_map=(0,))


def _lanesum(v):
    lanes = lax.iota(jnp.int32, 16)
    for d in (1, 2, 4, 8):
        idx = lax.bitwise_xor(lanes, jnp.full((16,), d, jnp.int32))
        v = v + lax.gather(v, idx[:, None], _GDN, (1,),
                           mode=lax.GatherScatterMode.PROMISE_IN_BOUNDS)
    return v


def _rsqrt16(x):
    s = 0.5 * (x + 1.0)
    for _ in range(14):
        s = 0.5 * (s + x / s)
    return 1.0 / s


@functools.partial(
    pl.kernel,
    mesh=_sc_mesh,
    out_type=jax.ShapeDtypeStruct((OUTR, DH), _f32),
    scratch_types=[
        pltpu.VMEM((16,), jnp.int32),
        pltpu.VMEM((CHUNK,), jnp.int32),
        pltpu.VMEM((CHUNK,), jnp.int32),
        pltpu.VMEM((CHUNK, DH), _f32),
        pltpu.VMEM((CHUNK, DH), _f32),
        pltpu.VMEM((CHUNK, DH), _f32),
        pltpu.VMEM((16, DH), _f32),
        pltpu.VMEM((2, DH), _f32),
        pltpu.VMEM((ACC_R, DH), _f32),
        pltpu.SemaphoreType.DMA,
        pltpu.SemaphoreType.DMA,
        pltpu.SemaphoreType.DMA,
        pltpu.SemaphoreType.DMA,
        pltpu.SemaphoreType.DMA,
    ],
)
def _edge_sc(A_hbm, B_hbm, C_hbm, src_hbm, dst_hbm, bnds_hbm, gbe_hbm,
             out_hbm, bv, src_v, dst_v, Av, Bv, Cv, tbuf, gbev, acc,
             semA, semB, semC, semS, semD):
    cid = lax.axis_index("c")
    sid = lax.axis_index("s")
    w = cid * 16 + sid

    pltpu.sync_copy(gbe_hbm, gbev)
    pltpu.sync_copy(bnds_hbm.at[pl.ds(w * 16, 16)], bv)
    bvec = bv[pl.ds(0, 16)]
    lo = bvec[0]
    hi = bvec[1]
    loa = (lo // 8) * 8
    niter = (hi - loa + (CHUNK - 1)) // CHUNK

    zero16 = jnp.zeros((16,), _f32)

    def _zrow(r, _):
        for k in range(NF):
            acc[r, pl.ds(k * 16, 16)] = zero16
        return 0

    lax.fori_loop(0, ACC_R, _zrow, 0)

    wbase = lax.broadcast_in_dim(w * RT, (16,), ())

    lane = lax.iota(jnp.int32, 16)

    def _group(g, _):
        dvec = dst_v[pl.ds(g * 16, 16)] - wbase
        svec = jnp.zeros((16,), _f32)
        qvec = jnp.zeros((16,), _f32)
        for j in range(16):
            e = g * 16 + j
            acc_s = jnp.zeros((16,), _f32)
            acc_q = jnp.zeros((16,), _f32)
            for k in range(NF):
                sl = pl.ds(k * 16, 16)
                t = Av[e, sl] + Bv[e, sl] + Cv[e, sl]
                tbuf[j, sl] = t
                acc_s = acc_s + t
                acc_q = acc_q + t * t
            jm = lane == j
            svec = jnp.where(jm, _lanesum(acc_s), svec)
            qvec = jnp.where(jm, _lanesum(acc_q), qvec)
        mvec = svec * (1.0 / DH)
        vvec = jnp.maximum(qvec * (1.0 / DH) - mvec * mvec, 0.0) + 1e-5
        ivec = _rsqrt16(vvec)
        for j in range(16):
            rl = dvec[j]
            ok = (rl >= 0) & (rl < RT)
            rl = jnp.where(ok, rl, RT)
            jidx = jnp.full((16,), j, jnp.int32)
            mj = lax.gather(mvec, jidx[:, None], _GDN, (1,),
                            mode=lax.GatherScatterMode.PROMISE_IN_BOUNDS)
            ij = lax.gather(ivec, jidx[:, None], _GDN, (1,),
                            mode=lax.GatherScatterMode.PROMISE_IN_BOUNDS)
            for k in range(NF):
                sl = pl.ds(k * 16, 16)
                y = (tbuf[j, sl] - mj) * ij
                y = jnp.maximum(y * gbev[0, sl] + gbev[1, sl], 0.0)
                acc[rl, sl] = acc[rl, sl] + y
        return 0

    def _step(jc, _):
        off = loa + jc * CHUNK
        cps = pltpu.async_copy(src_hbm.at[pl.ds(off, CHUNK)], src_v, semS)
        cpd = pltpu.async_copy(dst_hbm.at[pl.ds(off, CHUNK)], dst_v, semD)
        cpc = pltpu.async_copy(C_hbm.at[pl.ds(off, CHUNK)], Cv, semC)
        cps.wait()
        cpa = pltpu.async_copy(A_hbm.at[src_v], Av, semA)
        cpd.wait()
        cpb = pltpu.async_copy(B_hbm.at[dst_v], Bv, semB)
        cpc.wait()
        cpa.wait()
        cpb.wait()
        lax.fori_loop(0, NGRP, _group, 0)
        return 0

    lax.fori_loop(0, niter, _step, 0)
    pltpu.sync_copy(acc.at[pl.ds(0, RT)], out_hbm.at[pl.ds(w * RT, RT)])


def kernel(x, edge_index, edge_attr, pos, batch, emb_W, emb_b, emb_g, emb_be,
           node_W1, node_b1, node_g, node_be, node_W2, node_b2,
           edge_W1, edge_b1, edge_g, edge_be, edge_W2, edge_b2,
           upd_W, upd_b, ln_g, ln_be,
           out_W, out_b, out_g, out_be, comb_W, comb_b):
    src = edge_index[0]
    dst = edge_index[1]
    deg = jnp.zeros((N,), _f32).at[dst].add(1.0)

    order = jnp.argsort(dst)
    dst_s = jnp.pad(dst[order], (0, E_PAD - E), constant_values=N + 512)
    src_s = jnp.pad(src[order], (0, E_PAD - E))
    ea_s = jnp.pad(edge_attr[order], ((0, E_PAD - E), (0, 8 - DE)))
    bounds = jnp.searchsorted(dst_s[:E], jnp.arange(NW + 1, dtype=jnp.int32)
                              * RT).astype(jnp.int32)
    bnds = jnp.zeros((NW, 16), jnp.int32)
    bnds = bnds.at[:, 0].set(bounds[:NW]).at[:, 1].set(bounds[1:])
    bnds = bnds.reshape(-1)
    r1 = lambda v: v.reshape(1, -1)

    h = _emb(x, emb_W, r1(emb_b), r1(emb_g), r1(emb_be))
    for i in range(L):
        W1 = edge_W1[i]
        hn, A, B = _node_stage(h, node_W1[i], r1(node_b1[i]), r1(node_g[i]),
                               r1(node_be[i]), node_W2[i], r1(node_b2[i]),
                               W1[:DH], W1[DH:2 * DH])
        Wc = jnp.pad(W1[2 * DH:], ((0, 8 - DE), (0, 0)))
        C = _c_stage(ea_s, Wc, r1(edge_b1[i]))
        gbe = jnp.stack([edge_g[i], edge_be[i]])
        Sp = _edge_sc(A, B, C, src_s, dst_s, bnds, gbe)
        S = Sp[:N]
        dega = deg[:, None] * edge_b2[i][None, :]
        h = _upd_stage(S, dega, hn, h, edge_W2[i], upd_W[i][:DH],
                       upd_W[i][DH:], r1(upd_b[i]), r1(ln_g[i]), r1(ln_be[i]))

    node_emb = _out_stage(h, out_W, r1(out_b), r1(out_g), r1(out_be))
    graph_emb = _pool_stage(node_emb, batch.reshape(GRID_N, 1, BLK),
                            comb_W[:DOUT], comb_W[DOUT:], r1(comb_b))
    return node_emb, graph_emb


# speedup vs baseline: 1.3849x; 1.0922x over previous
import functools

import jax
import jax.numpy as jnp
from jax import lax
from jax.experimental import pallas as pl
from jax.experimental.pallas import tpu as pltpu
from jax.experimental.pallas import tpu_sc as plsc

N = 10000
E = 320000
DIN = 128
DH = 256
DOUT = 128
DE = 4
L = 3
G = 16

NW = 32
RT = 320
OUTR = NW * RT
ACC_R = RT + 8
CHUNK = 48
NGRP = CHUNK // 16
E_PAD = 323200
NF = DH // 16

BLK = 400
GRID_N = N // BLK

_f32 = jnp.float32


def _ln_rows(z, g, b):
    m = jnp.mean(z, axis=-1, keepdims=True)
    zc = z - m
    v = jnp.mean(zc * zc, axis=-1, keepdims=True)
    return zc * lax.rsqrt(v + 1e-5) * g + b


def _emb_body(x_ref, W_ref, b_ref, g_ref, be_ref, o_ref):
    z = jnp.dot(x_ref[:], W_ref[:], preferred_element_type=_f32) + b_ref[:]
    o_ref[:] = jnp.maximum(_ln_rows(z, g_ref[:], be_ref[:]), 0.0)


_emb = pl.pallas_call(
    _emb_body,
    grid=(GRID_N,),
    in_specs=[
        pl.BlockSpec((BLK, DIN), lambda i: (i, 0)),
        pl.BlockSpec((DIN, DH), lambda i: (0, 0)),
        pl.BlockSpec((1, DH), lambda i: (0, 0)),
        pl.BlockSpec((1, DH), lambda i: (0, 0)),
        pl.BlockSpec((1, DH), lambda i: (0, 0)),
    ],
    out_specs=pl.BlockSpec((BLK, DH), lambda i: (i, 0)),
    out_shape=jax.ShapeDtypeStruct((N, DH), _f32),
)


def _node_body(h_ref, nW1, nb1, ng, nbe, nW2, nb2, W1a, W1b,
               hn_ref, A_ref, B_ref):
    h = h_ref[:]
    z = jnp.dot(h, nW1[:], preferred_element_type=_f32) + nb1[:]
    h1 = jnp.maximum(_ln_rows(z, ng[:], nbe[:]), 0.0)
    hn_ref[:] = jnp.dot(h1, nW2[:], preferred_element_type=_f32) + nb2[:]
    A_ref[:] = jnp.dot(h, W1a[:], preferred_element_type=_f32)
    B_ref[:] = jnp.dot(h, W1b[:], preferred_element_type=_f32)


_mat_spec = pl.BlockSpec((DH, DH), lambda i: (0, 0))
_vec_spec = pl.BlockSpec((1, DH), lambda i: (0, 0))
_row_spec = pl.BlockSpec((BLK, DH), lambda i: (i, 0))

_node_stage = pl.pallas_call(
    _node_body,
    grid=(GRID_N,),
    in_specs=[_row_spec, _mat_spec, _vec_spec, _vec_spec, _vec_spec,
              _mat_spec, _vec_spec, _mat_spec, _mat_spec],
    out_specs=[_row_spec, _row_spec, _row_spec],
    out_shape=[jax.ShapeDtypeStruct((N, DH), _f32)] * 3,
)

EBLK = 3200
GRID_E = E_PAD // EBLK


def _c_body(ea_ref, Wc_ref, b1_ref, C_ref):
    C_ref[:] = (jnp.dot(ea_ref[:], Wc_ref[:], preferred_element_type=_f32)
                + b1_ref[:])


_c_stage = pl.pallas_call(
    _c_body,
    grid=(GRID_E,),
    in_specs=[
        pl.BlockSpec((EBLK, 8), lambda i: (i, 0)),
        pl.BlockSpec((8, DH), lambda i: (0, 0)),
        pl.BlockSpec((1, DH), lambda i: (0, 0)),
    ],
    out_specs=pl.BlockSpec((EBLK, DH), lambda i: (i, 0)),
    out_shape=jax.ShapeDtypeStruct((E_PAD, DH), _f32),
)


def _upd_body(S_ref, dega_ref, hn_ref, hp_ref, eW2, U1, U2, ub, lg, lb, o_ref):
    h_agg = jnp.dot(S_ref[:], eW2[:], preferred_element_type=_f32) + dega_ref[:]
    u = (jnp.dot(hn_ref[:], U1[:], preferred_element_type=_f32)
         + jnp.dot(h_agg, U2[:], preferred_element_type=_f32)
         + ub[:] + hp_ref[:])
    o_ref[:] = _ln_rows(u, lg[:], lb[:])


_upd_stage = pl.pallas_call(
    _upd_body,
    grid=(GRID_N,),
    in_specs=[_row_spec, _row_spec, _row_spec, _row_spec,
              _mat_spec, _mat_spec, _mat_spec, _vec_spec, _vec_spec, _vec_spec],
    out_specs=_row_spec,
    out_shape=jax.ShapeDtypeStruct((N, DH), _f32),
)


def _out_body(h_ref, W, b, g, be, o_ref):
    z = jnp.dot(h_ref[:], W[:], preferred_element_type=_f32) + b[:]
    o_ref[:] = _ln_rows(z, g[:], be[:])


_out_stage = pl.pallas_call(
    _out_body,
    grid=(GRID_N,),
    in_specs=[
        _row_spec,
        pl.BlockSpec((DH, DOUT), lambda i: (0, 0)),
        pl.BlockSpec((1, DOUT), lambda i: (0, 0)),
        pl.BlockSpec((1, DOUT), lambda i: (0, 0)),
        pl.BlockSpec((1, DOUT), lambda i: (0, 0)),
    ],
    out_specs=pl.BlockSpec((BLK, DOUT), lambda i: (i, 0)),
    out_shape=jax.ShapeDtypeStruct((N, DOUT), _f32),
)


def _pool_body(ne_ref, b_ref, Cw1, Cw2, cb, o_ref, acc, cacc):
    pid = pl.program_id(0)

    @pl.when(pid == 0)
    def _():
        acc[:] = jnp.zeros_like(acc)
        cacc[:] = jnp.zeros_like(cacc)

    bb = b_ref[0, 0, :]
    oh = (lax.broadcasted_iota(jnp.int32, (G, BLK), 0) == bb[None, :])
    oh = oh.astype(_f32)
    acc[:] += jnp.dot(oh, ne_ref[:], preferred_element_type=_f32)
    cacc[:] += jnp.broadcast_to(jnp.sum(oh, axis=1, keepdims=True), (G, DOUT))

    @pl.when(pid == GRID_N - 1)
    def _():
        a = acc[:]
        c = jnp.maximum(cacc[:], 1.0)
        o_ref[:] = (jnp.dot(a / c, Cw1[:], preferred_element_type=_f32)
                    + jnp.dot(a, Cw2[:], preferred_element_type=_f32) + cb[:])


_pool_stage = pl.pallas_call(
    _pool_body,
    grid=(GRID_N,),
    in_specs=[
        pl.BlockSpec((BLK, DOUT), lambda i: (i, 0)),
        pl.BlockSpec((1, 1, BLK), lambda i: (i, 0, 0)),
        pl.BlockSpec((DOUT, DOUT), lambda i: (0, 0)),
        pl.BlockSpec((DOUT, DOUT), lambda i: (0, 0)),
        pl.BlockSpec((1, DOUT), lambda i: (0, 0)),
    ],
    out_specs=pl.BlockSpec((G, DOUT), lambda i: (0, 0)),
    out_shape=jax.ShapeDtypeStruct((G, DOUT), _f32),
    scratch_shapes=[pltpu.VMEM((G, DOUT), _f32), pltpu.VMEM((G, DOUT), _f32)],
)


_sc_mesh = plsc.VectorSubcoreMesh(core_axis_name="c", subcore_axis_name="s")


_GDN = lax.GatherDimensionNumbers(
    offset_dims=(), collapsed_slice_dims=(0,), start_index_map=(0,))


def _lanesum(v):
    lanes = lax.iota(jnp.int32, 16)
    for d in (1, 2, 4, 8):
        idx = lax.bitwise_xor(lanes, jnp.full((16,), d, jnp.int32))
        v = v + lax.gather(v, idx[:, None], _GDN, (1,),
                           mode=lax.GatherScatterMode.PROMISE_IN_BOUNDS)
    return v


def _rsqrt16(x):
    s = 0.5 * (x + 1.0)
    for _ in range(14):
        s = 0.5 * (s + x / s)
    return 1.0 / s


@functools.partial(
    pl.kernel,
    mesh=_sc_mesh,
    out_type=jax.ShapeDtypeStruct((OUTR, DH), _f32),
    scratch_types=[
        pltpu.VMEM((16,), jnp.int32),
        pltpu.VMEM((CHUNK,), jnp.int32),
        pltpu.VMEM((CHUNK,), jnp.int32),
        pltpu.VMEM((CHUNK, DH), _f32),
        pltpu.VMEM((CHUNK, DH), _f32),
        pltpu.VMEM((CHUNK, DH), _f32),
        pltpu.VMEM((16, DH), _f32),
        pltpu.VMEM((2, DH), _f32),
        pltpu.VMEM((ACC_R, DH), _f32),
        pltpu.SemaphoreType.DMA,
        pltpu.SemaphoreType.DMA,
        pltpu.SemaphoreType.DMA,
        pltpu.SemaphoreType.DMA,
        pltpu.SemaphoreType.DMA,
    ],
)
def _edge_sc(A_hbm, B_hbm, C_hbm, src_hbm, dst_hbm, bnds_hbm, gbe_hbm,
             out_hbm, bv, src_v, dst_v, Av, Bv, Cv, tbuf, gbev, acc,
             semA, semB, semC, semS, semD):
    cid = lax.axis_index("c")
    sid = lax.axis_index("s")
    w = cid * 16 + sid

    pltpu.sync_copy(gbe_hbm, gbev)
    pltpu.sync_copy(bnds_hbm.at[pl.ds(w * 16, 16)], bv)
    bvec = bv[pl.ds(0, 16)]
    lo = bvec[0]
    hi = bvec[1]
    loa = (lo // 8) * 8
    niter = (hi - loa + (CHUNK - 1)) // CHUNK

    zero16 = jnp.zeros((16,), _f32)

    def _zrow(r, _):
        for k in range(NF):
            acc[r, pl.ds(k * 16, 16)] = zero16
        return 0

    lax.fori_loop(0, ACC_R, _zrow, 0)

    wbase = lax.broadcast_in_dim(w * RT, (16,), ())

    lane = lax.iota(jnp.int32, 16)
    gregs = [gbev[0, pl.ds(k * 16, 16)] for k in range(NF)]
    beregs = [gbev[1, pl.ds(k * 16, 16)] for k in range(NF)]

    def _group(g, _):
        dvec = dst_v[pl.ds(g * 16, 16)] - wbase
        svec = jnp.zeros((16,), _f32)
        qvec = jnp.zeros((16,), _f32)
        for j in range(16):
            e = g * 16 + j
            ss = [jnp.zeros((16,), _f32) for _ in range(4)]
            qq = [jnp.zeros((16,), _f32) for _ in range(4)]
            for k in range(NF):
                sl = pl.ds(k * 16, 16)
                t = Av[e, sl] + Bv[e, sl] + Cv[e, sl]
                tbuf[j, sl] = t
                ss[k % 4] = ss[k % 4] + t
                qq[k % 4] = qq[k % 4] + t * t
            acc_s = (ss[0] + ss[1]) + (ss[2] + ss[3])
            acc_q = (qq[0] + qq[1]) + (qq[2] + qq[3])
            jm = lane == j
            svec = jnp.where(jm, _lanesum(acc_s), svec)
            qvec = jnp.where(jm, _lanesum(acc_q), qvec)
        mvec = svec * (1.0 / DH)
        vvec = jnp.maximum(qvec * (1.0 / DH) - mvec * mvec, 0.0) + 1e-5
        ivec = _rsqrt16(vvec)
        for j in range(16):
            rl = dvec[j]
            ok = (rl >= 0) & (rl < RT)
            rl = jnp.where(ok, rl, RT)
            jidx = jnp.full((16,), j, jnp.int32)
            mj = lax.gather(mvec, jidx[:, None], _GDN, (1,),
                            mode=lax.GatherScatterMode.PROMISE_IN_BOUNDS)
            ij = lax.gather(ivec, jidx[:, None], _GDN, (1,),
                            mode=lax.GatherScatterMode.PROMISE_IN_BOUNDS)
            for k in range(NF):
                sl = pl.ds(k * 16, 16)
                y = (tbuf[j, sl] - mj) * ij
                y = jnp.maximum(y * gregs[k] + beregs[k], 0.0)
                acc[rl, sl] = acc[rl, sl] + y
        return 0

    def _step(jc, _):
        off = loa + jc * CHUNK
        cps = pltpu.async_copy(src_hbm.at[pl.ds(off, CHUNK)], src_v, semS)
        cpd = pltpu.async_copy(dst_hbm.at[pl.ds(off, CHUNK)], dst_v, semD)
        cpc = pltpu.async_copy(C_hbm.at[pl.ds(off, CHUNK)], Cv, semC)
        cps.wait()
        cpa = pltpu.async_copy(A_hbm.at[src_v], Av, semA)
        cpd.wait()
        cpb = pltpu.async_copy(B_hbm.at[dst_v], Bv, semB)
        cpc.wait()
        cpa.wait()
        cpb.wait()
        lax.fori_loop(0, NGRP, _group, 0)
        return 0

    lax.fori_loop(0, niter, _step, 0)
    pltpu.sync_copy(acc.at[pl.ds(0, RT)], out_hbm.at[pl.ds(w * RT, RT)])


def kernel(x, edge_index, edge_attr, pos, batch, emb_W, emb_b, emb_g, emb_be,
           node_W1, node_b1, node_g, node_be, node_W2, node_b2,
           edge_W1, edge_b1, edge_g, edge_be, edge_W2, edge_b2,
           upd_W, upd_b, ln_g, ln_be,
           out_W, out_b, out_g, out_be, comb_W, comb_b):
    src = edge_index[0]
    dst = edge_index[1]
    deg = jnp.zeros((N,), _f32).at[dst].add(1.0)

    order = jnp.argsort(dst)
    dst_s = jnp.pad(dst[order], (0, E_PAD - E), constant_values=N + 512)
    src_s = jnp.pad(src[order], (0, E_PAD - E))
    ea_s = jnp.pad(edge_attr[order], ((0, E_PAD - E), (0, 8 - DE)))
    bounds = jnp.searchsorted(dst_s[:E], jnp.arange(NW + 1, dtype=jnp.int32)
                              * RT).astype(jnp.int32)
    bnds = jnp.zeros((NW, 16), jnp.int32)
    bnds = bnds.at[:, 0].set(bounds[:NW]).at[:, 1].set(bounds[1:])
    bnds = bnds.reshape(-1)
    r1 = lambda v: v.reshape(1, -1)

    h = _emb(x, emb_W, r1(emb_b), r1(emb_g), r1(emb_be))
    for i in range(L):
        W1 = edge_W1[i]
        hn, A, B = _node_stage(h, node_W1[i], r1(node_b1[i]), r1(node_g[i]),
                               r1(node_be[i]), node_W2[i], r1(node_b2[i]),
                               W1[:DH], W1[DH:2 * DH])
        Wc = jnp.pad(W1[2 * DH:], ((0, 8 - DE), (0, 0)))
        C = _c_stage(ea_s, Wc, r1(edge_b1[i]))
        gbe = jnp.stack([edge_g[i], edge_be[i]])
        Sp = _edge_sc(A, B, C, src_s, dst_s, bnds, gbe)
        S = Sp[:N]
        dega = deg[:, None] * edge_b2[i][None, :]
        h = _upd_stage(S, dega, hn, h, edge_W2[i], upd_W[i][:DH],
                       upd_W[i][DH:], r1(upd_b[i]), r1(ln_g[i]), r1(ln_be[i]))

    node_emb = _out_stage(h, out_W, r1(out_b), r1(out_g), r1(out_be))
    graph_emb = _pool_stage(node_emb, batch.reshape(GRID_N, 1, BLK),
                            comb_W[:DOUT], comb_W[DOUT:], r1(comb_b))
    return node_emb, graph_emb
